# baseline (speedup 1.0000x reference)
import jax
import jax.numpy as jnp
from jax import lax
from jax.experimental import pallas as pl
from jax.experimental.pallas import tpu as pltpu
from jax.experimental.pallas import tpu_sc as plsc

_ROWS = 4096 * 50
_NC, _NS = 2, 16
_NW = _NC * _NS
_RPW = _ROWS // _NW
_R = 200
_NCHUNK = _RPW // _R

_mesh = plsc.VectorSubcoreMesh(core_axis_name="c", subcore_axis_name="s")

_out_t = jax.ShapeDtypeStruct((_ROWS, 2, 16), jnp.float32)


def _body(x_hbm, m_hbm, t_hbm, v_hbm, s_hbm,
          in0, in1, o00, o01, o02, o03, o10, o11, o12, o13,
          rs0, rs1, ws0, ws1):
    outs = (m_hbm, t_hbm, v_hbm, s_hbm)
    ins = (in0, in1)
    obufs = ((o00, o01, o02, o03), (o10, o11, o12, o13))
    rsems = (rs0, rs1)
    wsems = (ws0, ws1)
    wid = lax.axis_index("s") * _NC + lax.axis_index("c")
    base = wid * _RPW

    def start_read(c, b):
        pltpu.async_copy(x_hbm.at[pl.ds(base + c * _R, _R)], ins[b], rsems[b])

    def wait_read(b):
        pltpu.make_async_copy(x_hbm.at[pl.ds(0, _R)], ins[b], rsems[b]).wait()

    def start_writes(c, b):
        for k in range(4):
            pltpu.async_copy(
                obufs[b][k], outs[k].at[pl.ds(base + c * _R, _R)], wsems[b]
            )

    def wait_writes(b):
        for k in range(4):
            pltpu.make_async_copy(
                obufs[b][k], outs[k].at[pl.ds(0, _R)], wsems[b]
            ).wait()

    def rearrange(b):
        ib = ins[b]
        ob = obufs[b]

        def row(r, _):
            for i in range(8):
                ob[i // 2][r, i % 2, :] = ib[r, i, :]
            return ()

        lax.fori_loop(0, _R, row, (), unroll=4)

    start_read(0, 0)
    for c in range(_NCHUNK):
        b = c % 2
        nb = 1 - b
        if c + 1 < _NCHUNK:
            start_read(c + 1, nb)
        wait_read(b)
        if c >= 2:
            wait_writes(b)
        rearrange(b)
        start_writes(c, b)
    wait_writes(0)
    wait_writes(1)


_split = pl.kernel(
    _body,
    out_type=(_out_t,) * 4,
    mesh=_mesh,
    scratch_types=[pltpu.VMEM((_R, 8, 16), jnp.float32) for _ in range(2)]
    + [pltpu.VMEM((_R, 2, 16), jnp.float32) for _ in range(8)]
    + [pltpu.SemaphoreType.DMA for _ in range(4)],
    compiler_params=pltpu.CompilerParams(use_tc_tiling_on_sc=False),
)


@jax.jit
def kernel(x):
    xr = x.reshape(_ROWS, 8, 16)
    m, t, v, s = _split(xr)
    shp = (4096, 50, 32)
    return (m.reshape(shp), t.reshape(shp), v.reshape(shp), s.reshape(shp))


# speedup vs baseline: 1.0387x; 1.0387x over previous
import jax
import jax.numpy as jnp
from jax import lax
from jax.experimental import pallas as pl
from jax.experimental.pallas import tpu as pltpu
from jax.experimental.pallas import tpu_sc as plsc

_ROWS = 4096 * 50
_NC, _NS = 2, 16
_NW = _NC * _NS
_RPW = _ROWS // _NW
_R = 200
_NCHUNK = _RPW // _R

_mesh = plsc.VectorSubcoreMesh(core_axis_name="c", subcore_axis_name="s")

_out_t = jax.ShapeDtypeStruct((_ROWS, 2, 16), jnp.float32)


def _body(x_hbm, m_hbm, t_hbm, v_hbm, s_hbm,
          in0, in1, o00, o01, o02, o03, o10, o11, o12, o13,
          rs0, rs1, ws0, ws1):
    outs = (m_hbm, t_hbm, v_hbm, s_hbm)
    ins = (in0, in1)
    obufs = ((o00, o01, o02, o03), (o10, o11, o12, o13))
    rsems = (rs0, rs1)
    wsems = (ws0, ws1)
    wid = lax.axis_index("s") * _NC + lax.axis_index("c")
    base = wid * _RPW

    def start_read(c, b):
        pltpu.async_copy(x_hbm.at[pl.ds(base + c * _R, _R)], ins[b], rsems[b])

    def wait_read(b):
        pltpu.make_async_copy(x_hbm.at[pl.ds(0, _R)], ins[b], rsems[b]).wait()

    def start_writes(c, b):
        for k in range(4):
            pltpu.async_copy(
                obufs[b][k], outs[k].at[pl.ds(base + c * _R, _R)], wsems[b]
            )

    def wait_writes(b):
        for k in range(4):
            pltpu.make_async_copy(
                obufs[b][k], outs[k].at[pl.ds(0, _R)], wsems[b]
            ).wait()

    def rearrange(b):
        ib = ins[b]
        ob = obufs[b]

        @plsc.parallel_loop(0, _R, unroll=4)
        def row(r):
            for i in range(8):
                ob[i // 2][r, i % 2, :] = ib[r, i, :]

    start_read(0, 0)
    for c in range(_NCHUNK):
        b = c % 2
        nb = 1 - b
        if c + 1 < _NCHUNK:
            start_read(c + 1, nb)
        wait_read(b)
        if c >= 2:
            wait_writes(b)
        rearrange(b)
        start_writes(c, b)
    wait_writes(0)
    wait_writes(1)


_split = pl.kernel(
    _body,
    out_type=(_out_t,) * 4,
    mesh=_mesh,
    scratch_types=[pltpu.VMEM((_R, 8, 16), jnp.float32) for _ in range(2)]
    + [pltpu.VMEM((_R, 2, 16), jnp.float32) for _ in range(8)]
    + [pltpu.SemaphoreType.DMA for _ in range(4)],
    compiler_params=pltpu.CompilerParams(use_tc_tiling_on_sc=False),
)


@jax.jit
def kernel(x):
    xr = x.reshape(_ROWS, 8, 16)
    m, t, v, s = _split(xr)
    shp = (4096, 50, 32)
    return (m.reshape(shp), t.reshape(shp), v.reshape(shp), s.reshape(shp))


# speedup vs baseline: 4.9237x; 4.7402x over previous
import jax
import jax.numpy as jnp
from jax import lax
from jax.experimental import pallas as pl
from jax.experimental.pallas import tpu as pltpu
from jax.experimental.pallas import tpu_sc as plsc

_NBATCH = 4096
_NC, _NS = 2, 16
_NW = _NC * _NS
_BPW = _NBATCH // _NW
_B = 4
_NCHUNK = _BPW // _B

_mesh = plsc.VectorSubcoreMesh(core_axis_name="c", subcore_axis_name="s")

_out_t = jax.ShapeDtypeStruct((_NBATCH, 50, 32), jnp.float32)


def _body(x_hbm, m_hbm, t_hbm, v_hbm, s_hbm,
          in0, in1, o00, o01, o02, o03, o10, o11, o12, o13,
          rs0, rs1, ws0, ws1):
    outs = (m_hbm, t_hbm, v_hbm, s_hbm)
    ins = (in0, in1)
    obufs = ((o00, o01, o02, o03), (o10, o11, o12, o13))
    rsems = (rs0, rs1)
    wsems = (ws0, ws1)
    wid = lax.axis_index("s") * _NC + lax.axis_index("c")
    base = wid * _BPW

    def start_read(c, b):
        pltpu.async_copy(x_hbm.at[pl.ds(base + c * _B, _B)], ins[b], rsems[b])

    def wait_read(b):
        pltpu.make_async_copy(x_hbm.at[pl.ds(0, _B)], ins[b], rsems[b]).wait()

    def start_writes(c, b):
        for k in range(4):
            pltpu.async_copy(
                obufs[b][k], outs[k].at[pl.ds(base + c * _B, _B)], wsems[b]
            )

    def wait_writes(b):
        for k in range(4):
            pltpu.make_async_copy(
                obufs[b][k], outs[k].at[pl.ds(0, _B)], wsems[b]
            ).wait()

    def rearrange(b):
        ib = ins[b]
        ob = obufs[b]

        for bi in range(_B):
            @plsc.parallel_loop(0, 50, unroll=5)
            def row(t):
                for i in range(8):
                    ob[i // 2][bi, t, pl.ds(16 * (i % 2), 16)] = (
                        ib[bi, t, pl.ds(16 * i, 16)]
                    )

    start_read(0, 0)
    start_read(1, 1)

    def pair(i, _):
        for b in range(2):
            c = 2 * i + b
            wait_read(b)

            @pl.when(i > 0)
            def _():
                wait_writes(b)

            rearrange(b)
            start_writes(c, b)

            @pl.when(c + 2 < _NCHUNK)
            def _():
                start_read(c + 2, b)

        return ()

    lax.fori_loop(0, _NCHUNK // 2, pair, ())
    wait_writes(0)
    wait_writes(1)


_split = pl.kernel(
    _body,
    out_type=(_out_t,) * 4,
    mesh=_mesh,
    scratch_types=[pltpu.VMEM((_B, 50, 128), jnp.float32) for _ in range(2)]
    + [pltpu.VMEM((_B, 50, 32), jnp.float32) for _ in range(8)]
    + [pltpu.SemaphoreType.DMA for _ in range(4)],
    compiler_params=pltpu.CompilerParams(use_tc_tiling_on_sc=False),
)


@jax.jit
def kernel(x):
    return _split(x)


# speedup vs baseline: 6.4080x; 1.3015x over previous
import jax
import jax.numpy as jnp
from jax import lax
from jax.experimental import pallas as pl
from jax.experimental.pallas import tpu as pltpu
from jax.experimental.pallas import tpu_sc as plsc

_NB = 4096
_NT = 50
_NC, _NS = 2, 16
_NW = _NC * _NS
_BB = _NB // _NW

_mesh = plsc.VectorSubcoreMesh(core_axis_name="c", subcore_axis_name="s")

_y_t = jax.ShapeDtypeStruct((_NT, 4, _NW, 8, _BB), jnp.float32)


def _body(x_hbm, y0, y1, y2, y3,
          in0, in1, s00, s01, s02, s03, s10, s11, s12, s13,
          rs0, rs1, ws0, ws1):
    ys = (y0, y1, y2, y3)
    ins = (in0, in1)
    sts = ((s00, s01, s02, s03), (s10, s11, s12, s13))
    rsems = (rs0, rs1)
    wsems = (ws0, ws1)
    wid = lax.axis_index("s") * _NC + lax.axis_index("c")
    bbase = wid * _BB
    lane = lax.broadcasted_iota(jnp.int32, (16,), 0)

    def start_read(t, b):
        pltpu.async_copy(
            x_hbm.at[t, pl.ds(bbase, _BB)], ins[b], rsems[b]
        )

    def wait_read(b):
        pltpu.make_async_copy(
            x_hbm.at[0, pl.ds(0, _BB)], ins[b], rsems[b]
        ).wait()

    def start_writes(t, b):
        for k in range(4):
            pltpu.async_copy(sts[b][k], ys[k].at[t, :, wid], wsems[b])

    def wait_writes(b):
        for k in range(4):
            pltpu.make_async_copy(
                sts[b][k], ys[k].at[0, :, wid], wsems[b]
            ).wait()

    def transpose(b):
        ib = ins[b]
        for k in range(4):
            st = sts[b][k]

            @plsc.parallel_loop(0, 256, unroll=4)
            def vec(v):
                fb = v >> 6
                fr = (v >> 3) & 7
                blg = v & 7
                col = 32 * k + 8 * fb + fr
                rows = blg * 16 + lane
                cols = jnp.full((16,), 0, jnp.int32) + col
                st[fb, fr, pl.ds(blg * 16, 16)] = plsc.load_gather(
                    ib, [rows, cols]
                )

    start_read(0, 0)
    start_read(1, 1)

    def pair(i, _):
        for b in range(2):
            t = 2 * i + b
            wait_read(b)

            @pl.when(i > 0)
            def _():
                wait_writes(b)

            transpose(b)
            start_writes(t, b)

            @pl.when(t + 2 < _NT)
            def _():
                start_read(t + 2, b)

        return ()

    lax.fori_loop(0, _NT // 2, pair, ())
    wait_writes(0)
    wait_writes(1)


_split = pl.kernel(
    _body,
    out_type=(_y_t,) * 4,
    mesh=_mesh,
    scratch_types=[pltpu.VMEM((_BB, 128), jnp.float32) for _ in range(2)]
    + [pltpu.VMEM((4, 8, _BB), jnp.float32) for _ in range(8)]
    + [pltpu.SemaphoreType.DMA for _ in range(4)],
    compiler_params=pltpu.CompilerParams(
        use_tc_tiling_on_sc=False, needs_layout_passes=False
    ),
)


@jax.jit
def kernel(x):
    xt = x.transpose(1, 0, 2)
    ys = _split(xt)
    return tuple(
        y.transpose(2, 4, 0, 1, 3).reshape(_NB, _NT, 32) for y in ys
    )


# speedup vs baseline: 8.3702x; 1.3062x over previous
import jax
import jax.numpy as jnp
from jax import lax
from jax.experimental import pallas as pl
from jax.experimental.pallas import tpu as pltpu
from jax.experimental.pallas import tpu_sc as plsc

_NB = 4096
_NT = 50
_NC, _NS = 2, 16
_NW = _NC * _NS
_BB = _NB // _NW

_mesh = plsc.VectorSubcoreMesh(core_axis_name="c", subcore_axis_name="s")

_y_t = jax.ShapeDtypeStruct((_NT, 4, _NW, 8, _BB), jnp.float32)


def _body(x_hbm, y0, y1, y2, y3,
          in0, in1, s00, s01, s02, s03, s10, s11, s12, s13,
          rs0, rs1, ws0, ws1):
    ys = (y0, y1, y2, y3)
    ins = (in0, in1)
    sts = ((s00, s01, s02, s03), (s10, s11, s12, s13))
    rsems = (rs0, rs1)
    wsems = (ws0, ws1)
    wid = lax.axis_index("s") * _NC + lax.axis_index("c")
    bbase = wid * _BB
    lane = lax.broadcasted_iota(jnp.int32, (16,), 0)

    def start_read(t, b):
        pltpu.async_copy(
            x_hbm.at[t, pl.ds(bbase * 128, _BB * 128)], ins[b], rsems[b]
        )

    def wait_read(b):
        pltpu.make_async_copy(
            x_hbm.at[0, pl.ds(0, _BB * 128)], ins[b], rsems[b]
        ).wait()

    def start_writes(t, b):
        for k in range(4):
            pltpu.async_copy(sts[b][k], ys[k].at[t, :, wid], wsems[b])

    def wait_writes(b):
        for k in range(4):
            pltpu.make_async_copy(
                sts[b][k], ys[k].at[0, :, wid], wsems[b]
            ).wait()

    def transpose(b):
        for k in range(4):
            st = sts[b][k]
            ib = ins[b].at[pl.ds(32 * k, 16288)]
            for blg in range(8):
                rowoff = (blg * 16 + lane) * 128

                @plsc.parallel_loop(0, 32, unroll=4)
                def vec(cp):
                    fb = cp >> 3
                    fr = cp & 7
                    idx = rowoff + jnp.full((16,), 0, jnp.int32) + cp
                    st[fb, fr, pl.ds(blg * 16, 16)] = plsc.load_gather(
                        ib, [idx]
                    )

    start_read(0, 0)
    start_read(1, 1)

    def pair(i, _):
        for b in range(2):
            t = 2 * i + b
            wait_read(b)

            @pl.when(i > 0)
            def _():
                wait_writes(b)

            transpose(b)
            start_writes(t, b)

            @pl.when(t + 2 < _NT)
            def _():
                start_read(t + 2, b)

        return ()

    lax.fori_loop(0, _NT // 2, pair, ())
    wait_writes(0)
    wait_writes(1)


_split = pl.kernel(
    _body,
    out_type=(_y_t,) * 4,
    mesh=_mesh,
    scratch_types=[pltpu.VMEM((_BB * 128,), jnp.float32) for _ in range(2)]
    + [pltpu.VMEM((4, 8, _BB), jnp.float32) for _ in range(8)]
    + [pltpu.SemaphoreType.DMA for _ in range(4)],
    compiler_params=pltpu.CompilerParams(
        use_tc_tiling_on_sc=False, needs_layout_passes=False
    ),
)


@jax.jit
def kernel(x):
    xt = x.transpose(1, 0, 2).reshape(_NT, _NB * 128)
    ys = _split(xt)
    return tuple(
        y.transpose(2, 4, 0, 1, 3).reshape(_NB, _NT, 32) for y in ys
    )


# speedup vs baseline: 18.8397x; 2.2508x over previous
import jax
import jax.numpy as jnp
from jax import lax
from jax.experimental import pallas as pl
from jax.experimental.pallas import tpu as pltpu
from jax.experimental.pallas import tpu_sc as plsc

_NB = 4096
_NT = 50
_NC, _NS = 2, 16
_NW = _NC * _NS
_BB = _NB // _NW

_mesh = plsc.VectorSubcoreMesh(core_axis_name="c", subcore_axis_name="s")

_y_t = jax.ShapeDtypeStruct((_NT, 4, _NW, 8, _BB), jnp.float32)


def _body(x_hbm, y0, y1, y2, y3,
          in0, in1, s00, s01, s02, s03, s10, s11, s12, s13,
          rs0, rs1, ws0, ws1):
    ys = (y0, y1, y2, y3)
    ins = (in0, in1)
    sts = ((s00, s01, s02, s03), (s10, s11, s12, s13))
    rsems = (rs0, rs1)
    wsems = (ws0, ws1)
    wid = lax.axis_index("s") * _NC + lax.axis_index("c")
    bbase = wid * _BB
    lane = lax.broadcasted_iota(jnp.int32, (16,), 0)
    rots = [(lane + d) & 15 for d in range(16)]
    gidx = [[lane * 128 + cg * 16 + r for r in rots] for cg in range(2)]
    fbidx = [[(cg * 16 + r) >> 3 for r in rots] for cg in range(2)]
    fridx = [r & 7 for r in rots]

    def start_read(t, b):
        pltpu.async_copy(
            x_hbm.at[t, pl.ds(bbase * 128, _BB * 128)], ins[b], rsems[b]
        )

    def wait_read(b):
        pltpu.make_async_copy(
            x_hbm.at[0, pl.ds(0, _BB * 128)], ins[b], rsems[b]
        ).wait()

    def start_writes(t, b):
        for k in range(4):
            pltpu.async_copy(sts[b][k], ys[k].at[t, :, wid], wsems[b])

    def wait_writes(b):
        for k in range(4):
            pltpu.make_async_copy(
                sts[b][k], ys[k].at[0, :, wid], wsems[b]
            ).wait()

    def transpose(b):
        ib = ins[b]
        for k in range(4):
            st = sts[b][k]
            for cg in range(2):
                @plsc.parallel_loop(0, 8)
                def blk(blg):
                    base = 32 * k + blg * 2048
                    blidx = blg * 16 + lane
                    for d in range(16):
                        val = plsc.load_gather(ib, [gidx[cg][d] + base])
                        plsc.store_scatter(
                            st, [fbidx[cg][d], fridx[d], blidx], val
                        )

    start_read(0, 0)
    start_read(1, 1)

    def pair(i, _):
        for b in range(2):
            t = 2 * i + b
            wait_read(b)

            @pl.when(i > 0)
            def _():
                wait_writes(b)

            transpose(b)
            start_writes(t, b)

            @pl.when(t + 2 < _NT)
            def _():
                start_read(t + 2, b)

        return ()

    lax.fori_loop(0, _NT // 2, pair, ())
    wait_writes(0)
    wait_writes(1)


_split = pl.kernel(
    _body,
    out_type=(_y_t,) * 4,
    mesh=_mesh,
    scratch_types=[pltpu.VMEM((_BB * 128,), jnp.float32) for _ in range(2)]
    + [pltpu.VMEM((4, 8, _BB), jnp.float32) for _ in range(8)]
    + [pltpu.SemaphoreType.DMA for _ in range(4)],
    compiler_params=pltpu.CompilerParams(
        use_tc_tiling_on_sc=False, needs_layout_passes=False
    ),
)


@jax.jit
def kernel(x):
    xt = x.transpose(1, 0, 2).reshape(_NT, _NB * 128)
    ys = _split(xt)
    return tuple(
        y.transpose(2, 4, 0, 1, 3).reshape(_NB, _NT, 32) for y in ys
    )
